# baseline (speedup 1.0000x reference)
import numpy as np
import jax
import jax.numpy as jnp
from jax import lax
from jax.experimental import pallas as pl
from jax.experimental.pallas import tpu as pltpu
from jax.experimental.pallas import tpu_sc as plsc

_NUM_CLASSES = 100000
_FEAT = 128
_BATCH = 16384
_BIA = 0.4

_NC = 2
_NS = 16
_NW = _NC * _NS

_B_PER_W = _BATCH // _NW
_B_CHUNK = 64
_B_NCHUNK = _B_PER_W // _B_CHUNK

_PAIRS = _NUM_CLASSES // 2
_P_PER_W = 1568
_PAIRS_PAD = _P_PER_W * _NW
_P_CHUNK = 112
_P_NCHUNK = _P_PER_W // _P_CHUNK
_P_GROUPS = _P_CHUNK // 16


def _pair_indices():
    np.random.seed(0)
    perm = np.random.permutation(_NUM_CLASSES)
    pad = np.arange(_PAIRS_PAD - _PAIRS, dtype=np.int32)
    idx_a = np.concatenate([perm[:_PAIRS].astype(np.int32), pad])
    idx_b = np.concatenate([perm[_PAIRS:].astype(np.int32), pad])
    return jnp.asarray(idx_a), jnp.asarray(idx_b)


_GATHER_DNUMS = lax.GatherDimensionNumbers(
    offset_dims=(), collapsed_slice_dims=(0,), start_index_map=(0,))


def _lane_rot(v, k):
    idx = (lax.iota(jnp.int32, 16) + k) & 15
    return lax.gather(v, idx[:, None], _GATHER_DNUMS, (1,),
                      mode=lax.GatherScatterMode.PROMISE_IN_BOUNDS)


def _sc_body(label_h, feat_h, centers_h, idxa_h, idxb_h,
             part_h, norms_h,
             lbl_v, cent0_v, cent1_v, feat0_v, feat1_v,
             idxa_v, idxb_v, a0_v, a1_v, b0_v, b1_v, norms_v,
             part_v, sems):
    wid = lax.axis_index("s") * _NC + lax.axis_index("c")
    base = wid * _B_PER_W
    pbase = wid * _P_PER_W

    cents = (cent0_v, cent1_v)
    feats = (feat0_v, feat1_v)
    avs = (a0_v, a1_v)
    bvs = (b0_v, b1_v)

    pltpu.sync_copy(label_h.at[pl.ds(base, _B_PER_W)], lbl_v)

    def start_p1(ci):
        slot = ci % 2
        hc = pltpu.async_copy(
            centers_h.at[lbl_v.at[pl.ds(ci * _B_CHUNK, _B_CHUNK)]],
            cents[slot], sems[slot])
        hf = pltpu.async_copy(
            feat_h.at[pl.ds(base + ci * _B_CHUNK, _B_CHUNK)],
            feats[slot], sems[2 + slot])
        return hc, hf

    def start_p2(ci):
        slot = ci % 2
        ha = pltpu.async_copy(
            centers_h.at[idxa_v.at[pl.ds(ci * _P_CHUNK, _P_CHUNK)]],
            avs[slot], sems[4 + slot])
        hb = pltpu.async_copy(
            centers_h.at[idxb_v.at[pl.ds(ci * _P_CHUNK, _P_CHUNK)]],
            bvs[slot], sems[6 + slot])
        return ha, hb

    p1_handles = [start_p1(0), start_p1(1)]
    pltpu.sync_copy(idxa_h.at[pl.ds(pbase, _P_PER_W)], idxa_v)
    pltpu.sync_copy(idxb_h.at[pl.ds(pbase, _P_PER_W)], idxb_v)
    p2_handles = [start_p2(0), start_p2(1)]
    accs = tuple(jnp.zeros((16,), jnp.float32) for _ in range(8))
    for ci in range(_B_NCHUNK):
        slot = ci % 2
        hc, hf = p1_handles[slot]
        hc.wait()
        hf.wait()
        if ci + 2 < _B_NCHUNK:
            p1_handles[slot] = start_p1(ci + 2)
        cent_v = cents[slot]
        feat_v = feats[slot]

        def row_body(r, a):
            out = []
            for j in range(8):
                f = feat_v[r, pl.ds(j * 16, 16)]
                c = cent_v[r, pl.ds(j * 16, 16)]
                d = f - c
                out.append(a[j] + d * d)
            return tuple(out)

        accs = lax.fori_loop(0, _B_CHUNK, row_body, accs)
    total = accs[0] * jnp.float32(0.01)
    for j in range(1, 8):
        total = total + accs[j]
    part_v[...] = total
    pltpu.sync_copy(part_v, part_h.at[wid])

    lane = lax.iota(jnp.int32, 16)
    for ci in range(_P_NCHUNK):
        slot = ci % 2
        ha, hb = p2_handles[slot]
        ha.wait()
        hb.wait()
        if ci + 2 < _P_NCHUNK:
            p2_handles[slot] = start_p2(ci + 2)
        a_v = avs[slot]
        b_v = bvs[slot]

        def grp_body(g, _):
            r = jnp.zeros((16,), jnp.float32)
            for pj in range(16):
                p = g * 16 + pj
                d0 = a_v[p, pl.ds(0, 16)] - b_v[p, pl.ds(0, 16)]
                d1 = a_v[p, pl.ds(16, 16)] - b_v[p, pl.ds(16, 16)]
                s = d0 * d0 + d1 * d1
                for k in (8, 4, 2, 1):
                    s = s + _lane_rot(s, k)
                r = jnp.where(lane == pj, s, r)
            norms_v[pl.ds(ci * _P_CHUNK + g * 16, 16)] = r
            return 0

        lax.fori_loop(0, _P_GROUPS, grp_body, 0)
    pltpu.sync_copy(norms_v, norms_h.at[pl.ds(pbase, _P_PER_W)])


def _fin_body(part_ref, norms_ref, loss_ref, dloss_ref, dist_ref):
    psum = jnp.sum(part_ref[...])
    loss = psum * jnp.float32(0.5 / _BATCH)
    dist = jnp.sum(jnp.sqrt(norms_ref[...])) * jnp.float32(1.0 / _PAIRS)
    loss_ref[...] = loss.reshape(1, 1)
    dist_ref[...] = dist.reshape(1, 1)
    dloss_ref[...] = (jnp.float32(1.0) / (dist + jnp.float32(_BIA))).reshape(1, 1)


def kernel(label, feat, centers):
    idxa, idxb = _pair_indices()

    mesh = plsc.VectorSubcoreMesh(core_axis_name="c", subcore_axis_name="s")
    sc = pl.kernel(
        _sc_body,
        mesh=mesh,
        out_type=[
            jax.ShapeDtypeStruct((_NW, 16), jnp.float32),
            jax.ShapeDtypeStruct((_PAIRS_PAD,), jnp.float32),
        ],
        scratch_types=[
            pltpu.VMEM((_B_PER_W,), jnp.int32),
            pltpu.VMEM((_B_CHUNK, _FEAT), jnp.float32),
            pltpu.VMEM((_B_CHUNK, _FEAT), jnp.float32),
            pltpu.VMEM((_B_CHUNK, _FEAT), jnp.float32),
            pltpu.VMEM((_B_CHUNK, _FEAT), jnp.float32),
            pltpu.VMEM((_P_PER_W,), jnp.int32),
            pltpu.VMEM((_P_PER_W,), jnp.int32),
            pltpu.VMEM((_P_CHUNK, _FEAT), jnp.float32),
            pltpu.VMEM((_P_CHUNK, _FEAT), jnp.float32),
            pltpu.VMEM((_P_CHUNK, _FEAT), jnp.float32),
            pltpu.VMEM((_P_CHUNK, _FEAT), jnp.float32),
            pltpu.VMEM((_P_PER_W,), jnp.float32),
            pltpu.VMEM((16,), jnp.float32),
            [pltpu.SemaphoreType.DMA] * 8,
        ],
    )
    part, norms2 = sc(label.astype(jnp.int32), feat, centers, idxa, idxb)

    loss2d, dloss2d, dist2d = pl.pallas_call(
        _fin_body,
        out_shape=[
            jax.ShapeDtypeStruct((1, 1), jnp.float32),
            jax.ShapeDtypeStruct((1, 1), jnp.float32),
            jax.ShapeDtypeStruct((1, 1), jnp.float32),
        ],
    )(part.reshape(4, 128), norms2.reshape(_PAIRS_PAD // 128, 128))

    return (loss2d.reshape(()), dloss2d.reshape(()), dist2d.reshape(()))


# speedup vs baseline: 1.0580x; 1.0580x over previous
import numpy as np
import jax
import jax.numpy as jnp
from jax import lax
from jax.experimental import pallas as pl
from jax.experimental.pallas import tpu as pltpu
from jax.experimental.pallas import tpu_sc as plsc

_NUM_CLASSES = 100000
_FEAT = 128
_BATCH = 16384
_BIA = 0.4

_NC = 2
_NS = 16
_NW = _NC * _NS

_B_PER_W = _BATCH // _NW
_B_CHUNK = 64
_B_NCHUNK = _B_PER_W // _B_CHUNK

_PAIRS = _NUM_CLASSES // 2
_P_PER_W = 1600
_PAIRS_PAD = _P_PER_W * _NW
_P_CHUNK = 160
_P_NCHUNK = _P_PER_W // _P_CHUNK
_P_GROUPS = _P_CHUNK // 16


def _pair_indices():
    np.random.seed(0)
    perm = np.random.permutation(_NUM_CLASSES)
    pad = np.arange(_PAIRS_PAD - _PAIRS, dtype=np.int32)
    idx_a = np.concatenate([perm[:_PAIRS].astype(np.int32), pad])
    idx_b = np.concatenate([perm[_PAIRS:].astype(np.int32), pad])
    return jnp.asarray(idx_a), jnp.asarray(idx_b)


_GATHER_DNUMS = lax.GatherDimensionNumbers(
    offset_dims=(), collapsed_slice_dims=(0,), start_index_map=(0,))


def _lane_rot(v, k):
    idx = (lax.iota(jnp.int32, 16) + k) & 15
    return lax.gather(v, idx[:, None], _GATHER_DNUMS, (1,),
                      mode=lax.GatherScatterMode.PROMISE_IN_BOUNDS)


def _sc_body(label_h, feat_h, centers_h, idxa_h, idxb_h,
             part_h, norms_h,
             lbl_v, cent0_v, cent1_v, feat0_v, feat1_v,
             idxa_v, idxb_v, a0_v, a1_v, b0_v, b1_v, norms_v,
             part_v, sems):
    wid = lax.axis_index("s") * _NC + lax.axis_index("c")
    base = wid * _B_PER_W
    pbase = wid * _P_PER_W

    cents = (cent0_v, cent1_v)
    feats = (feat0_v, feat1_v)
    avs = (a0_v, a1_v)
    bvs = (b0_v, b1_v)

    pltpu.sync_copy(label_h.at[pl.ds(base, _B_PER_W)], lbl_v)

    def start_p1(ci):
        slot = ci % 2
        hc = pltpu.async_copy(
            centers_h.at[lbl_v.at[pl.ds(ci * _B_CHUNK, _B_CHUNK)]],
            cents[slot], sems[slot])
        hf = pltpu.async_copy(
            feat_h.at[pl.ds(base + ci * _B_CHUNK, _B_CHUNK)],
            feats[slot], sems[2 + slot])
        return hc, hf

    def start_p2(ci):
        slot = ci % 2
        ha = pltpu.async_copy(
            centers_h.at[idxa_v.at[pl.ds(ci * _P_CHUNK, _P_CHUNK)]],
            avs[slot], sems[4 + slot])
        hb = pltpu.async_copy(
            centers_h.at[idxb_v.at[pl.ds(ci * _P_CHUNK, _P_CHUNK)]],
            bvs[slot], sems[6 + slot])
        return ha, hb

    p1_handles = [start_p1(0), start_p1(1)]
    pltpu.sync_copy(idxa_h.at[pl.ds(pbase, _P_PER_W)], idxa_v)
    pltpu.sync_copy(idxb_h.at[pl.ds(pbase, _P_PER_W)], idxb_v)
    p2_handles = [start_p2(0), start_p2(1)]
    accs = tuple(jnp.zeros((16,), jnp.float32) for _ in range(8))
    for ci in range(_B_NCHUNK):
        slot = ci % 2
        hc, hf = p1_handles[slot]
        hc.wait()
        hf.wait()
        if ci + 2 < _B_NCHUNK:
            p1_handles[slot] = start_p1(ci + 2)
        cent_v = cents[slot]
        feat_v = feats[slot]

        def row_body(r, a):
            out = []
            for j in range(8):
                f = feat_v[r, pl.ds(j * 16, 16)]
                c = cent_v[r, pl.ds(j * 16, 16)]
                d = f - c
                out.append(a[j] + d * d)
            return tuple(out)

        accs = lax.fori_loop(0, _B_CHUNK, row_body, accs)
    total = accs[0] * jnp.float32(0.01)
    for j in range(1, 8):
        total = total + accs[j]
    part_v[...] = total
    pltpu.sync_copy(part_v, part_h.at[wid])

    lane = lax.iota(jnp.int32, 16)
    for ci in range(_P_NCHUNK):
        slot = ci % 2
        ha, hb = p2_handles[slot]
        ha.wait()
        hb.wait()
        if ci + 2 < _P_NCHUNK:
            p2_handles[slot] = start_p2(ci + 2)
        a_v = avs[slot]
        b_v = bvs[slot]

        def grp_body(g, _):
            r = jnp.zeros((16,), jnp.float32)
            for pj in range(16):
                p = g * 16 + pj
                d0 = a_v[p, pl.ds(0, 16)] - b_v[p, pl.ds(0, 16)]
                d1 = a_v[p, pl.ds(16, 16)] - b_v[p, pl.ds(16, 16)]
                s = d0 * d0 + d1 * d1
                for k in (8, 4, 2, 1):
                    s = s + _lane_rot(s, k)
                r = jnp.where(lane == pj, s, r)
            norms_v[pl.ds(ci * _P_CHUNK + g * 16, 16)] = r
            return 0

        lax.fori_loop(0, _P_GROUPS, grp_body, 0)
    pltpu.sync_copy(norms_v, norms_h.at[pl.ds(pbase, _P_PER_W)])


def _fin_body(part_ref, norms_ref, loss_ref, dloss_ref, dist_ref):
    psum = jnp.sum(part_ref[...])
    loss = psum * jnp.float32(0.5 / _BATCH)
    dist = jnp.sum(jnp.sqrt(norms_ref[...])) * jnp.float32(1.0 / _PAIRS)
    loss_ref[...] = loss.reshape(1, 1)
    dist_ref[...] = dist.reshape(1, 1)
    dloss_ref[...] = (jnp.float32(1.0) / (dist + jnp.float32(_BIA))).reshape(1, 1)


def kernel(label, feat, centers):
    idxa, idxb = _pair_indices()

    mesh = plsc.VectorSubcoreMesh(core_axis_name="c", subcore_axis_name="s")
    sc = pl.kernel(
        _sc_body,
        mesh=mesh,
        out_type=[
            jax.ShapeDtypeStruct((_NW, 16), jnp.float32),
            jax.ShapeDtypeStruct((_PAIRS_PAD,), jnp.float32),
        ],
        scratch_types=[
            pltpu.VMEM((_B_PER_W,), jnp.int32),
            pltpu.VMEM((_B_CHUNK, _FEAT), jnp.float32),
            pltpu.VMEM((_B_CHUNK, _FEAT), jnp.float32),
            pltpu.VMEM((_B_CHUNK, _FEAT), jnp.float32),
            pltpu.VMEM((_B_CHUNK, _FEAT), jnp.float32),
            pltpu.VMEM((_P_PER_W,), jnp.int32),
            pltpu.VMEM((_P_PER_W,), jnp.int32),
            pltpu.VMEM((_P_CHUNK, _FEAT), jnp.float32),
            pltpu.VMEM((_P_CHUNK, _FEAT), jnp.float32),
            pltpu.VMEM((_P_CHUNK, _FEAT), jnp.float32),
            pltpu.VMEM((_P_CHUNK, _FEAT), jnp.float32),
            pltpu.VMEM((_P_PER_W,), jnp.float32),
            pltpu.VMEM((16,), jnp.float32),
            [pltpu.SemaphoreType.DMA] * 8,
        ],
    )
    part, norms2 = sc(label.astype(jnp.int32), feat, centers, idxa, idxb)

    loss2d, dloss2d, dist2d = pl.pallas_call(
        _fin_body,
        out_shape=[
            jax.ShapeDtypeStruct((1, 1), jnp.float32),
            jax.ShapeDtypeStruct((1, 1), jnp.float32),
            jax.ShapeDtypeStruct((1, 1), jnp.float32),
        ],
    )(part.reshape(4, 128), norms2.reshape(_PAIRS_PAD // 128, 128))

    return (loss2d.reshape(()), dloss2d.reshape(()), dist2d.reshape(()))
